# baseline (speedup 1.0000x reference)
import jax
import jax.numpy as jnp
from jax import lax
from jax.experimental import pallas as pl
from jax.experimental.pallas import tpu as pltpu
from jax.experimental.pallas import tpu_sc as plsc

N, K, D = 32768, 1024, 64
HALF = N // 2
ROWS_PER_BLOCK = 512
NUM_BLOCKS = N // ROWS_PER_BLOCK
HALF_BLOCKS = NUM_BLOCKS // 2
NW = 32
P_PER_W = HALF // NW
IDX_CHUNK = 128
CHUNKS = P_PER_W // IDX_CHUNK


def _argmax_body(x_ref, idx_ref):
    xb = x_ref[...]
    m = jnp.max(xb, axis=-1, keepdims=True)
    col = lax.broadcasted_iota(jnp.int32, xb.shape, 1)
    cand = jnp.where(xb == m, col, K)
    am = jnp.min(cand, axis=-1)
    idx_ref[0, 0] = am.reshape(CHUNKS, IDX_CHUNK)


def _tc_argmax(x):
    return pl.pallas_call(
        _argmax_body,
        grid=(NUM_BLOCKS,),
        in_specs=[pl.BlockSpec((ROWS_PER_BLOCK, K), lambda b: (b, 0))],
        out_specs=pl.BlockSpec(
            (1, 1, CHUNKS, IDX_CHUNK),
            lambda b: (b % HALF_BLOCKS, b // HALF_BLOCKS, 0, 0),
        ),
        out_shape=jax.ShapeDtypeStruct((NW, 2, CHUNKS, IDX_CHUNK), jnp.int32),
    )(x)


def _sc_gather_body(table_hbm, idx_hbm, out_hbm, idx_v, lo_v, hi_v, sem):
    wid = lax.axis_index("s") * 2 + lax.axis_index("c")
    pltpu.sync_copy(idx_hbm.at[wid], idx_v)
    for p, buf in ((0, lo_v), (1, hi_v)):
        for j in range(CHUNKS):
            pltpu.async_copy(
                table_hbm.at[idx_v.at[p, j]],
                buf.at[pl.ds(j * IDX_CHUNK, IDX_CHUNK)],
                sem,
            ).wait()
    base = wid * P_PER_W
    pltpu.sync_copy(lo_v, out_hbm.at[pl.ds(base, P_PER_W), pl.ds(0, D)])
    pltpu.sync_copy(hi_v, out_hbm.at[pl.ds(base, P_PER_W), pl.ds(D, D)])


def _sc_gather(table, idx4):
    mesh = plsc.VectorSubcoreMesh(core_axis_name="c", subcore_axis_name="s")
    run = pl.kernel(
        _sc_gather_body,
        out_type=jax.ShapeDtypeStruct((HALF, 2 * D), jnp.float32),
        mesh=mesh,
        scratch_types=[
            pltpu.VMEM((2, CHUNKS, IDX_CHUNK), jnp.int32),
            pltpu.VMEM((P_PER_W, D), jnp.float32),
            pltpu.VMEM((P_PER_W, D), jnp.float32),
            pltpu.SemaphoreType.DMA,
        ],
        compiler_params=pltpu.CompilerParams(use_tc_tiling_on_sc=False),
    )
    return run(table, idx4)


_FMT_ROWS = 2048


def _fmt_body(in_ref, out_ref):
    blk = in_ref[...]
    out_ref[0] = blk[:, :D]
    out_ref[1] = blk[:, D:]


def _tc_format(out2):
    out3 = pl.pallas_call(
        _fmt_body,
        grid=(HALF // _FMT_ROWS,),
        in_specs=[pl.BlockSpec((_FMT_ROWS, 2 * D), lambda i: (i, 0))],
        out_specs=pl.BlockSpec((2, _FMT_ROWS, D), lambda i: (0, i, 0)),
        out_shape=jax.ShapeDtypeStruct((2, HALF, D), jnp.float32),
    )(out2)
    return out3.reshape(N, D)


def kernel(x, W):
    table = jnp.transpose(W)
    idx4 = _tc_argmax(x)
    out2 = _sc_gather(table, idx4)
    return _tc_format(out2)


# speedup vs baseline: 1.1753x; 1.1753x over previous
import jax
import jax.numpy as jnp
from jax import lax
from jax.experimental import pallas as pl
from jax.experimental.pallas import tpu as pltpu
from jax.experimental.pallas import tpu_sc as plsc

N, K, D = 32768, 1024, 64
HALF = N // 2
ROWS_PER_BLOCK = 1024
NUM_BLOCKS = N // ROWS_PER_BLOCK
HALF_BLOCKS = NUM_BLOCKS // 2
W_PER_BLOCK = 2
NW = 32
P_PER_W = HALF // NW
IDX_CHUNK = 128
CHUNKS = P_PER_W // IDX_CHUNK


def _argmax_body(x_ref, idx_ref):
    xb = x_ref[...]
    m = jnp.max(xb, axis=-1, keepdims=True)
    col = lax.broadcasted_iota(jnp.int32, xb.shape, 1)
    cand = jnp.where(xb == m, col, K)
    am = jnp.min(cand, axis=-1)
    idx_ref[:, 0] = am.reshape(W_PER_BLOCK, CHUNKS, IDX_CHUNK)


def _tc_argmax(x):
    return pl.pallas_call(
        _argmax_body,
        grid=(NUM_BLOCKS,),
        in_specs=[pl.BlockSpec((ROWS_PER_BLOCK, K), lambda b: (b, 0))],
        out_specs=pl.BlockSpec(
            (W_PER_BLOCK, 1, CHUNKS, IDX_CHUNK),
            lambda b: (b % HALF_BLOCKS, b // HALF_BLOCKS, 0, 0),
        ),
        out_shape=jax.ShapeDtypeStruct((NW, 2, CHUNKS, IDX_CHUNK), jnp.int32),
    )(x)


def _sc_gather_body(table_hbm, idx_hbm, out_hbm, idx_v, lo_v, hi_v, sem):
    wid = lax.axis_index("s") * 2 + lax.axis_index("c")
    pltpu.sync_copy(idx_hbm.at[wid], idx_v)
    for p, buf in ((0, lo_v), (1, hi_v)):
        for j in range(CHUNKS):
            pltpu.async_copy(
                table_hbm.at[idx_v.at[p, j]],
                buf.at[pl.ds(j * IDX_CHUNK, IDX_CHUNK)],
                sem,
            ).wait()
    base = wid * P_PER_W
    pltpu.sync_copy(lo_v, out_hbm.at[pl.ds(base, P_PER_W), pl.ds(0, D)])
    pltpu.sync_copy(hi_v, out_hbm.at[pl.ds(base, P_PER_W), pl.ds(D, D)])


def _sc_gather(table, idx4):
    mesh = plsc.VectorSubcoreMesh(core_axis_name="c", subcore_axis_name="s")
    run = pl.kernel(
        _sc_gather_body,
        out_type=jax.ShapeDtypeStruct((HALF, 2 * D), jnp.float32),
        mesh=mesh,
        scratch_types=[
            pltpu.VMEM((2, CHUNKS, IDX_CHUNK), jnp.int32),
            pltpu.VMEM((P_PER_W, D), jnp.float32),
            pltpu.VMEM((P_PER_W, D), jnp.float32),
            pltpu.SemaphoreType.DMA,
        ],
        compiler_params=pltpu.CompilerParams(use_tc_tiling_on_sc=False),
    )
    return run(table, idx4)


_FMT_ROWS = 2048


def _fmt_body(in_ref, out_ref):
    blk = in_ref[...]
    out_ref[0] = blk[:, :D]
    out_ref[1] = blk[:, D:]


def _tc_format(out2):
    out3 = pl.pallas_call(
        _fmt_body,
        grid=(HALF // _FMT_ROWS,),
        in_specs=[pl.BlockSpec((_FMT_ROWS, 2 * D), lambda i: (i, 0))],
        out_specs=pl.BlockSpec((2, _FMT_ROWS, D), lambda i: (0, i, 0)),
        out_shape=jax.ShapeDtypeStruct((2, HALF, D), jnp.float32),
    )(out2)
    return out3.reshape(N, D)


def kernel(x, W):
    table = jnp.transpose(W)
    idx4 = _tc_argmax(x)
    out2 = _sc_gather(table, idx4)
    return _tc_format(out2)


# speedup vs baseline: 1.2047x; 1.0250x over previous
import jax
import jax.numpy as jnp
from jax import lax
from jax.experimental import pallas as pl
from jax.experimental.pallas import tpu as pltpu
from jax.experimental.pallas import tpu_sc as plsc

N, K, D = 32768, 1024, 64
HALF = N // 2
QUARTER = N // 4
ROWS_PER_BLOCK = 1024
BLOCKS_PER_HALF = HALF // ROWS_PER_BLOCK
NW = 32
T_PER_W = HALF // NW
P_PER_W = T_PER_W // 2
IDX_CHUNK = 128
CHUNKS = P_PER_W // IDX_CHUNK
W_PER_BLOCK = ROWS_PER_BLOCK // P_PER_W


def _argmax_body(x_ref, idx_ref):
    xb = x_ref[...]
    m = jnp.max(xb, axis=-1, keepdims=True)
    col = lax.broadcasted_iota(jnp.int32, xb.shape, 1)
    cand = jnp.where(xb == m, col, K)
    am = jnp.min(cand, axis=-1)
    idx_ref[:, 0] = am.reshape(W_PER_BLOCK, CHUNKS, IDX_CHUNK)


def _tc_argmax_half(x, h):
    return pl.pallas_call(
        _argmax_body,
        grid=(BLOCKS_PER_HALF,),
        in_specs=[
            pl.BlockSpec(
                (ROWS_PER_BLOCK, K),
                lambda b, h=h: (h * BLOCKS_PER_HALF + b, 0),
            )
        ],
        out_specs=pl.BlockSpec(
            (W_PER_BLOCK, 1, CHUNKS, IDX_CHUNK),
            lambda b: (b % (BLOCKS_PER_HALF // 2), b // (BLOCKS_PER_HALF // 2), 0, 0),
        ),
        out_shape=jax.ShapeDtypeStruct((NW, 2, CHUNKS, IDX_CHUNK), jnp.int32),
    )(x)


def _sc_gather_body(table_hbm, idx_hbm, out_hbm, idx_v, lo_v, hi_v, sem):
    wid = lax.axis_index("s") * 2 + lax.axis_index("c")
    pltpu.sync_copy(idx_hbm.at[wid], idx_v)
    for p, buf in ((0, lo_v), (1, hi_v)):
        for j in range(CHUNKS):
            pltpu.async_copy(
                table_hbm.at[idx_v.at[p, j]],
                buf.at[pl.ds(j * IDX_CHUNK, IDX_CHUNK)],
                sem,
            ).wait()
    base = wid * P_PER_W
    pltpu.sync_copy(lo_v, out_hbm.at[pl.ds(base, P_PER_W), pl.ds(0, D)])
    pltpu.sync_copy(hi_v, out_hbm.at[pl.ds(base, P_PER_W), pl.ds(D, D)])


def _sc_gather(table, idx4):
    mesh = plsc.VectorSubcoreMesh(core_axis_name="c", subcore_axis_name="s")
    run = pl.kernel(
        _sc_gather_body,
        out_type=jax.ShapeDtypeStruct((QUARTER, 2 * D), jnp.float32),
        mesh=mesh,
        scratch_types=[
            pltpu.VMEM((2, CHUNKS, IDX_CHUNK), jnp.int32),
            pltpu.VMEM((P_PER_W, D), jnp.float32),
            pltpu.VMEM((P_PER_W, D), jnp.float32),
            pltpu.SemaphoreType.DMA,
        ],
        compiler_params=pltpu.CompilerParams(use_tc_tiling_on_sc=False),
    )
    return run(table, idx4)


_FMT_ROWS = 2048


def _fmt_body(lo_ref, hi_ref, out_ref):
    lo = lo_ref[...]
    hi = hi_ref[...]
    out_ref[0] = lo[:, :D]
    out_ref[1] = lo[:, D:]
    out_ref[2] = hi[:, :D]
    out_ref[3] = hi[:, D:]


def _tc_format(lo2, hi2):
    out4 = pl.pallas_call(
        _fmt_body,
        grid=(QUARTER // _FMT_ROWS,),
        in_specs=[
            pl.BlockSpec((_FMT_ROWS, 2 * D), lambda i: (i, 0)),
            pl.BlockSpec((_FMT_ROWS, 2 * D), lambda i: (i, 0)),
        ],
        out_specs=pl.BlockSpec((4, _FMT_ROWS, D), lambda i: (0, i, 0)),
        out_shape=jax.ShapeDtypeStruct((4, QUARTER, D), jnp.float32),
    )(lo2, hi2)
    return out4.reshape(N, D)


def kernel(x, W):
    table = jnp.transpose(W)
    idx_lo = _tc_argmax_half(x, 0)
    lo2 = _sc_gather(table, idx_lo)
    idx_hi = _tc_argmax_half(x, 1)
    hi2 = _sc_gather(table, idx_hi)
    return _tc_format(lo2, hi2)


# speedup vs baseline: 1.3717x; 1.1386x over previous
import jax
import jax.numpy as jnp
from jax import lax
from jax.experimental import pallas as pl
from jax.experimental.pallas import tpu as pltpu
from jax.experimental.pallas import tpu_sc as plsc

N, K, D = 32768, 1024, 64
ROWS_PER_BLOCK = 1024
NUM_BLOCKS = N // ROWS_PER_BLOCK
NW = 32
CG = N // 128
DG = D // 8


def _argmax_body(x_ref, idx_ref):
    xb = x_ref[...]
    m = jnp.max(xb, axis=-1, keepdims=True)
    col = lax.broadcasted_iota(jnp.int32, xb.shape, 1)
    cand = jnp.where(xb == m, col, K)
    am = jnp.min(cand, axis=-1)
    idx_ref[...] = am.reshape(ROWS_PER_BLOCK // 128, 128)


def _tc_argmax(x):
    return pl.pallas_call(
        _argmax_body,
        grid=(NUM_BLOCKS,),
        in_specs=[pl.BlockSpec((ROWS_PER_BLOCK, K), lambda b: (b, 0))],
        out_specs=pl.BlockSpec((ROWS_PER_BLOCK // 128, 128), lambda b: (b, 0)),
        out_shape=jax.ShapeDtypeStruct((CG, 128), jnp.int32),
    )(x)


def _sc_gather_body(w_hbm, idx_hbm, out_hbm, w_v, idx_v, outv0, outv1, sem):
    wid = lax.axis_index("s") * 2 + lax.axis_index("c")
    d0 = wid * 2
    rg = wid // 4
    s0 = (wid % 4) * 2
    pltpu.sync_copy(w_hbm.at[pl.ds(d0, 2)], w_v)
    pltpu.sync_copy(idx_hbm, idx_v)
    row0 = jnp.zeros((16,), jnp.int32)
    row1 = row0 + 1

    def cg_body(cg, carry):
        for g in range(8):
            tok = idx_v[cg, pl.ds(g * 16, 16)]
            v0 = plsc.load_gather(w_v, [row0, tok])
            v1 = plsc.load_gather(w_v, [row1, tok])
            outv0[cg, 0, pl.ds(g * 16, 16)] = v0
            outv1[cg, 0, pl.ds(g * 16, 16)] = v1
        return carry

    lax.fori_loop(0, CG, cg_body, 0)
    pltpu.sync_copy(outv0, out_hbm.at[rg, :, pl.ds(s0, 1), :])
    pltpu.sync_copy(outv1, out_hbm.at[rg, :, pl.ds(s0 + 1, 1), :])


def _sc_gather(W, idx2):
    mesh = plsc.VectorSubcoreMesh(core_axis_name="c", subcore_axis_name="s")
    run = pl.kernel(
        _sc_gather_body,
        out_type=jax.ShapeDtypeStruct((DG, CG, 8, 128), jnp.float32),
        mesh=mesh,
        scratch_types=[
            pltpu.VMEM((2, K), jnp.float32),
            pltpu.VMEM((CG, 128), jnp.int32),
            pltpu.VMEM((CG, 1, 128), jnp.float32),
            pltpu.VMEM((CG, 1, 128), jnp.float32),
            pltpu.SemaphoreType.DMA,
        ],
        compiler_params=pltpu.CompilerParams(
            use_tc_tiling_on_sc=False, needs_layout_passes=False
        ),
    )
    return run(W, idx2)


def kernel(x, W):
    idx2 = _tc_argmax(x)
    out4 = _sc_gather(W, idx2)
    return out4.transpose(0, 2, 1, 3).reshape(D, N).T
